# initial kernel scaffold (unmeasured)
import jax
import jax.numpy as jnp
from jax import lax
from jax.experimental import pallas as pl
from jax.experimental.pallas import tpu as pltpu

N_DEV = 32
B = 64
D = 1024
ROWS = B // N_DEV


def kernel(x, Win0, Wout0, Win1, Wout1, Win2, Wout2):
    def body(x_ref, win0, wout0, win1, wout1, win2, wout2, out_ref,
             partial_buf, rs_buf, red_buf, xcur,
             rs_send, rs_recv, ag_send, ag_recv):
        my = lax.axis_index("i")

        def peer(k):
            return lax.rem(my + k, N_DEV)

        def rs_send_desc(k):
            dst = peer(k)
            return pltpu.make_async_remote_copy(
                src_ref=partial_buf.at[dst],
                dst_ref=rs_buf.at[my],
                send_sem=rs_send.at[dst],
                recv_sem=rs_recv.at[my],
                device_id=(dst,),
                device_id_type=pl.DeviceIdType.MESH,
            )

        def rs_recv_desc(k):
            p = peer(k)
            return pltpu.make_async_remote_copy(
                src_ref=partial_buf.at[p],
                dst_ref=rs_buf.at[p],
                send_sem=rs_send.at[p],
                recv_sem=rs_recv.at[p],
                device_id=(p,),
                device_id_type=pl.DeviceIdType.MESH,
            )

        def ag_send_desc(k):
            dst = peer(k)
            return pltpu.make_async_remote_copy(
                src_ref=red_buf,
                dst_ref=xcur.at[my],
                send_sem=ag_send.at[dst],
                recv_sem=ag_recv.at[my],
                device_id=(dst,),
                device_id_type=pl.DeviceIdType.MESH,
            )

        def ag_recv_desc(k):
            p = peer(k)
            return pltpu.make_async_remote_copy(
                src_ref=red_buf,
                dst_ref=xcur.at[p],
                send_sem=ag_send.at[p],
                recv_sem=ag_recv.at[p],
                device_id=(p,),
                device_id_type=pl.DeviceIdType.MESH,
            )

        def loop(f):
            lax.fori_loop(1, N_DEV, lambda k, c: (f(k), c)[1], 0)

        def layer(xall, win, wout):
            h = lax.dot(
                xall.astype(jnp.bfloat16),
                win[...].astype(jnp.bfloat16),
                preferred_element_type=jnp.float32,
            )
            h = jnp.maximum(h, 0.0)
            p = lax.dot(
                h.astype(jnp.bfloat16),
                wout[...].astype(jnp.bfloat16),
                preferred_element_type=jnp.float32,
            )
            partial_buf[...] = p.reshape(N_DEV, ROWS, D)
            rs_buf[pl.ds(my, 1)] = partial_buf[pl.ds(my, 1)]

            loop(lambda k: rs_send_desc(k).start())
            loop(lambda k: rs_recv_desc(k).wait_recv())

            reduced = jnp.sum(rs_buf[...], axis=0)
            red_buf[...] = reduced
            xcur[pl.ds(my, 1)] = reduced[None]

            loop(lambda k: ag_send_desc(k).start())
            loop(lambda k: ag_recv_desc(k).wait_recv())

            loop(lambda k: rs_send_desc(k).wait_send())
            loop(lambda k: ag_send_desc(k).wait_send())
            return xcur[...].reshape(B, D)

        xall = x_ref[...]
        xall = layer(xall, win0, wout0)
        xall = layer(xall, win1, wout1)
        xall = layer(xall, win2, wout2)
        out_ref[...] = xall

    out_shape = jax.ShapeDtypeStruct((B, D), jnp.float32)
    return pl.pallas_call(
        body,
        out_shape=out_shape,
        in_specs=[pl.BlockSpec(memory_space=pltpu.VMEM)] * 7,
        out_specs=pl.BlockSpec(memory_space=pltpu.VMEM),
        scratch_shapes=[
            pltpu.VMEM((N_DEV, ROWS, D), jnp.float32),
            pltpu.VMEM((N_DEV, ROWS, D), jnp.float32),
            pltpu.VMEM((ROWS, D), jnp.float32),
            pltpu.VMEM((N_DEV, ROWS, D), jnp.float32),
            pltpu.SemaphoreType.DMA((N_DEV,)),
            pltpu.SemaphoreType.DMA((N_DEV,)),
            pltpu.SemaphoreType.DMA((N_DEV,)),
            pltpu.SemaphoreType.DMA((N_DEV,)),
        ],
        compiler_params=pltpu.CompilerParams(has_side_effects=True),
    )(x, Win0, Wout0, Win1, Wout1, Win2, Wout2)


# baseline (device time: 72763 ns/iter reference)
import jax
import jax.numpy as jnp
from jax import lax
from jax.experimental import pallas as pl
from jax.experimental.pallas import tpu as pltpu

N_DEV = 32
B = 64
D = 1024
ROWS = B // N_DEV


def kernel(x, Win0, Wout0, Win1, Wout1, Win2, Wout2):
    def body(x_ref, win0, wout0, win1, wout1, win2, wout2, out_ref,
             partial_buf, rs_buf, red_buf, xcur,
             rs_send, rs_recv, ag_send, ag_recv):
        my = lax.axis_index("i")

        def peer(k):
            return lax.rem(my + k, N_DEV)

        def rs_send_desc(k):
            dst = peer(k)
            return pltpu.make_async_remote_copy(
                src_ref=partial_buf.at[dst],
                dst_ref=rs_buf.at[my],
                send_sem=rs_send.at[dst],
                recv_sem=rs_recv.at[my],
                device_id=(dst,),
                device_id_type=pl.DeviceIdType.MESH,
            )

        def rs_recv_desc(k):
            p = peer(k)
            return pltpu.make_async_remote_copy(
                src_ref=partial_buf.at[p],
                dst_ref=rs_buf.at[p],
                send_sem=rs_send.at[p],
                recv_sem=rs_recv.at[p],
                device_id=(p,),
                device_id_type=pl.DeviceIdType.MESH,
            )

        def ag_send_desc(k):
            dst = peer(k)
            return pltpu.make_async_remote_copy(
                src_ref=red_buf,
                dst_ref=xcur.at[my],
                send_sem=ag_send.at[dst],
                recv_sem=ag_recv.at[my],
                device_id=(dst,),
                device_id_type=pl.DeviceIdType.MESH,
            )

        def ag_recv_desc(k):
            p = peer(k)
            return pltpu.make_async_remote_copy(
                src_ref=red_buf,
                dst_ref=xcur.at[p],
                send_sem=ag_send.at[p],
                recv_sem=ag_recv.at[p],
                device_id=(p,),
                device_id_type=pl.DeviceIdType.MESH,
            )

        def loop(f):
            lax.fori_loop(1, N_DEV, lambda k, c: (f(k), c)[1], 0)

        def layer(xall, win, wout):
            h = lax.dot(
                xall.astype(jnp.bfloat16),
                win[...].astype(jnp.bfloat16),
                preferred_element_type=jnp.float32,
            )
            h = jnp.maximum(h, 0.0)
            p = lax.dot(
                h.astype(jnp.bfloat16),
                wout[...].astype(jnp.bfloat16),
                preferred_element_type=jnp.float32,
            )
            partial_buf[...] = p.reshape(N_DEV, ROWS, D)
            rs_buf[pl.ds(my, 1)] = partial_buf[pl.ds(my, 1)]

            loop(lambda k: rs_send_desc(k).start())
            loop(lambda k: rs_recv_desc(k).wait_recv())

            reduced = jnp.sum(rs_buf[...], axis=0)
            red_buf[...] = reduced
            xcur[pl.ds(my, 1)] = reduced[None]

            loop(lambda k: ag_send_desc(k).start())
            loop(lambda k: ag_recv_desc(k).wait_recv())

            loop(lambda k: rs_send_desc(k).wait_send())
            loop(lambda k: ag_send_desc(k).wait_send())
            return xcur[...].reshape(B, D)

        xall = x_ref[...]
        xall = layer(xall, win0, wout0)
        xall = layer(xall, win1, wout1)
        xall = layer(xall, win2, wout2)
        out_ref[...] = xall

    out_shape = jax.ShapeDtypeStruct((B, D), jnp.float32)
    return pl.pallas_call(
        body,
        out_shape=out_shape,
        in_specs=[pl.BlockSpec(memory_space=pltpu.VMEM)] * 7,
        out_specs=pl.BlockSpec(memory_space=pltpu.VMEM),
        scratch_shapes=[
            pltpu.VMEM((N_DEV, ROWS, D), jnp.float32),
            pltpu.VMEM((N_DEV, ROWS, D), jnp.float32),
            pltpu.VMEM((ROWS, D), jnp.float32),
            pltpu.VMEM((N_DEV, ROWS, D), jnp.float32),
            pltpu.SemaphoreType.DMA((N_DEV,)),
            pltpu.SemaphoreType.DMA((N_DEV,)),
            pltpu.SemaphoreType.DMA((N_DEV,)),
            pltpu.SemaphoreType.DMA((N_DEV,)),
        ],
        compiler_params=pltpu.CompilerParams(
            has_side_effects=True,
            vmem_limit_bytes=100 * 1024 * 1024,
        ),
    )(x, Win0, Wout0, Win1, Wout1, Win2, Wout2)


# device time: 53185 ns/iter; 1.3681x vs baseline; 1.3681x over previous
import jax
import jax.numpy as jnp
from jax import lax
from jax.experimental import pallas as pl
from jax.experimental.pallas import tpu as pltpu

N_DEV = 32
B = 64
D = 1024
H = 2048
ROWS = B // N_DEV

_BF = jnp.bfloat16


def kernel(x, Win0, Wout0, Win1, Wout1, Win2, Wout2):
    def body(x_ref, win0, wout0, win1, wout1, win2, wout2, out_ref,
             partial_buf, rs_buf, red_buf, xcur,
             stage_win, stage_wout, win_bf, wout_bf,
             load_sems, rs_send, rs_recv, ag_send, ag_recv):
        my = lax.axis_index("i")
        wins = [win0, win1, win2]
        wouts = [wout0, wout1, wout2]

        def peer(k):
            return lax.rem(my + k, N_DEV)

        def load_descs(l):
            return (
                pltpu.make_async_copy(wins[l], stage_win, load_sems.at[0]),
                pltpu.make_async_copy(wouts[l], stage_wout, load_sems.at[1]),
            )

        def start_load(l):
            a, b = load_descs(l)
            a.start()
            b.start()

        def wait_load(l):
            a, b = load_descs(l)
            a.wait()
            b.wait()

        def convert(slot):
            win_bf[slot] = stage_win[...].astype(_BF)
            wout_bf[slot] = stage_wout[...].astype(_BF)

        def rs_send_desc(k):
            dst = peer(k)
            return pltpu.make_async_remote_copy(
                src_ref=partial_buf.at[dst],
                dst_ref=rs_buf.at[my],
                send_sem=rs_send.at[dst],
                recv_sem=rs_recv.at[my],
                device_id=(dst,),
                device_id_type=pl.DeviceIdType.MESH,
            )

        def rs_recv_desc(k):
            p = peer(k)
            return pltpu.make_async_remote_copy(
                src_ref=partial_buf.at[p],
                dst_ref=rs_buf.at[p],
                send_sem=rs_send.at[p],
                recv_sem=rs_recv.at[p],
                device_id=(p,),
                device_id_type=pl.DeviceIdType.MESH,
            )

        def ag_send_desc(k):
            dst = peer(k)
            return pltpu.make_async_remote_copy(
                src_ref=red_buf,
                dst_ref=xcur.at[my],
                send_sem=ag_send.at[dst],
                recv_sem=ag_recv.at[my],
                device_id=(dst,),
                device_id_type=pl.DeviceIdType.MESH,
            )

        def ag_recv_desc(k):
            p = peer(k)
            return pltpu.make_async_remote_copy(
                src_ref=red_buf,
                dst_ref=xcur.at[p],
                send_sem=ag_send.at[p],
                recv_sem=ag_recv.at[p],
                device_id=(p,),
                device_id_type=pl.DeviceIdType.MESH,
            )

        def loop(f):
            lax.fori_loop(1, N_DEV, lambda k, c: (f(k), c)[1], 0)

        barrier = pltpu.get_barrier_semaphore()
        loop(lambda k: pl.semaphore_signal(
            barrier, inc=1,
            device_id=(peer(k),), device_id_type=pl.DeviceIdType.MESH,
        ))

        def layer(l, xall):
            slot = l % 2
            h = lax.dot(xall, win_bf[slot], preferred_element_type=jnp.float32)
            h = jnp.maximum(h, 0.0).astype(_BF)
            p = lax.dot(h, wout_bf[slot], preferred_element_type=jnp.float32)
            partial_buf[...] = p.astype(_BF).reshape(N_DEV, ROWS, D)
            rs_buf[pl.ds(my, 1)] = partial_buf[pl.ds(my, 1)]

            if l == 0:
                pl.semaphore_wait(barrier, N_DEV - 1)

            loop(lambda k: rs_send_desc(k).start())
            if l < 2:
                wait_load(l + 1)
                convert(1 - slot)
                if l == 0:
                    start_load(2)
            loop(lambda k: rs_recv_desc(k).wait_recv())

            reduced = jnp.sum(
                rs_buf[...].astype(jnp.float32), axis=0
            )
            red_buf[...] = reduced.astype(_BF)
            xcur[pl.ds(my, 1)] = red_buf[...][None]

            loop(lambda k: ag_send_desc(k).start())
            loop(lambda k: rs_send_desc(k).wait_send())
            loop(lambda k: ag_recv_desc(k).wait_recv())
            loop(lambda k: ag_send_desc(k).wait_send())
            return xcur[...].reshape(B, D)

        start_load(0)
        wait_load(0)
        convert(0)
        start_load(1)
        xall = x_ref[...].astype(_BF)
        xall = layer(0, xall)
        xall = layer(1, xall)
        xall = layer(2, xall)
        out_ref[...] = xall.astype(jnp.float32)

    out_shape = jax.ShapeDtypeStruct((B, D), jnp.float32)
    return pl.pallas_call(
        body,
        out_shape=out_shape,
        in_specs=[pl.BlockSpec(memory_space=pltpu.VMEM)]
        + [pl.BlockSpec(memory_space=pltpu.MemorySpace.HBM)] * 6,
        out_specs=pl.BlockSpec(memory_space=pltpu.VMEM),
        scratch_shapes=[
            pltpu.VMEM((N_DEV, ROWS, D), _BF),
            pltpu.VMEM((N_DEV, ROWS, D), _BF),
            pltpu.VMEM((ROWS, D), _BF),
            pltpu.VMEM((N_DEV, ROWS, D), _BF),
            pltpu.VMEM((D, H), jnp.float32),
            pltpu.VMEM((H, D), jnp.float32),
            pltpu.VMEM((2, D, H), _BF),
            pltpu.VMEM((2, H, D), _BF),
            pltpu.SemaphoreType.DMA((2,)),
            pltpu.SemaphoreType.DMA((N_DEV,)),
            pltpu.SemaphoreType.DMA((N_DEV,)),
            pltpu.SemaphoreType.DMA((N_DEV,)),
            pltpu.SemaphoreType.DMA((N_DEV,)),
        ],
        compiler_params=pltpu.CompilerParams(
            has_side_effects=True,
            collective_id=0,
            vmem_limit_bytes=100 * 1024 * 1024,
        ),
    )(x, Win0, Wout0, Win1, Wout1, Win2, Wout2)


# device time: 50750 ns/iter; 1.4338x vs baseline; 1.0480x over previous
import jax
import jax.numpy as jnp
from jax import lax
from jax.experimental import pallas as pl
from jax.experimental.pallas import tpu as pltpu

N_DEV = 32
B = 64
D = 1024
H = 2048
ROWS = B // N_DEV
GROUPS = 4
PER_G = N_DEV // GROUPS

_BF = jnp.bfloat16


def kernel(x, Win0, Wout0, Win1, Wout1, Win2, Wout2):
    def body(x_ref, win0, wout0, win1, wout1, win2, wout2, out_ref,
             partial_buf, rs_buf, red_buf, xcur,
             stage_win, stage_wout, win_bf, wout_bf,
             load_sems, rs_send, rs_recv, ag_send, ag_recv):
        my = lax.axis_index("i")
        wins = [win0, win1, win2]
        wouts = [wout0, wout1, wout2]

        def peer(k):
            return lax.rem(my + k, N_DEV)

        def win_load(l):
            return pltpu.make_async_copy(wins[l], stage_win, load_sems.at[0])

        def wout_load(l):
            return pltpu.make_async_copy(wouts[l], stage_wout, load_sems.at[1])

        def rs_send_desc(k):
            dst = peer(k)
            return pltpu.make_async_remote_copy(
                src_ref=partial_buf.at[dst],
                dst_ref=rs_buf.at[my],
                send_sem=rs_send.at[dst],
                recv_sem=rs_recv.at[my],
                device_id=(dst,),
                device_id_type=pl.DeviceIdType.MESH,
            )

        def rs_recv_desc(k):
            p = peer(k)
            return pltpu.make_async_remote_copy(
                src_ref=partial_buf.at[p],
                dst_ref=rs_buf.at[p],
                send_sem=rs_send.at[p],
                recv_sem=rs_recv.at[p],
                device_id=(p,),
                device_id_type=pl.DeviceIdType.MESH,
            )

        def ag_send_desc(k):
            dst = peer(k)
            return pltpu.make_async_remote_copy(
                src_ref=red_buf,
                dst_ref=xcur.at[my],
                send_sem=ag_send.at[dst],
                recv_sem=ag_recv.at[my],
                device_id=(dst,),
                device_id_type=pl.DeviceIdType.MESH,
            )

        def ag_recv_desc(p):
            return pltpu.make_async_remote_copy(
                src_ref=red_buf,
                dst_ref=xcur.at[p],
                send_sem=ag_send.at[p],
                recv_sem=ag_recv.at[p],
                device_id=(p,),
                device_id_type=pl.DeviceIdType.MESH,
            )

        def loop(f):
            lax.fori_loop(1, N_DEV, lambda k, c: (f(k), c)[1], 0)

        barrier = pltpu.get_barrier_semaphore()
        loop(lambda k: pl.semaphore_signal(
            barrier, inc=1,
            device_id=(peer(k),), device_id_type=pl.DeviceIdType.MESH,
        ))

        def layer(l, xall, last):
            slot = l % 2
            h = lax.dot(xall, win_bf[slot], preferred_element_type=jnp.float32)
            h = jnp.maximum(h, 0.0).astype(_BF)
            p = lax.dot(h, wout_bf[slot], preferred_element_type=jnp.float32)
            partial_buf[...] = p.astype(_BF).reshape(N_DEV, ROWS, D)

            if l == 0:
                pl.semaphore_wait(barrier, N_DEV - 1)

            loop(lambda k: rs_send_desc(k).start())
            rs_buf[pl.ds(my, 1)] = partial_buf[pl.ds(my, 1)]
            if not last:
                win_load(l + 1).wait()
                win_bf[1 - slot] = stage_win[...].astype(_BF)
                if l == 0:
                    win_load(2).start()
            loop(lambda k: rs_recv_desc(k).wait_recv())

            reduced = jnp.sum(
                rs_buf[...].astype(jnp.float32), axis=0
            )
            red_buf[...] = reduced.astype(_BF)
            xcur[pl.ds(my, 1)] = red_buf[...][None]

            loop(lambda k: ag_send_desc(k).start())
            loop(lambda k: rs_send_desc(k).wait_send())
            if not last:
                wout_load(l + 1).wait()
                wout_bf[1 - slot] = stage_wout[...].astype(_BF)
                if l == 0:
                    wout_load(2).start()

            if last:
                for g in range(GROUPS):
                    for c in range(g * PER_G, (g + 1) * PER_G):
                        @pl.when(c != my)
                        def _wait():
                            ag_recv_desc(c).wait_recv()
                    lo = g * PER_G
                    out_ref[pl.ds(lo * ROWS, PER_G * ROWS), :] = (
                        xcur[lo:lo + PER_G].reshape(PER_G * ROWS, D)
                        .astype(jnp.float32)
                    )
            else:
                loop(lambda k: ag_recv_desc(peer(k)).wait_recv())
            loop(lambda k: ag_send_desc(k).wait_send())
            return None if last else xcur[...].reshape(B, D)

        win_load(0).start()
        wout_load(0).start()
        xall = x_ref[...].astype(_BF)
        win_load(0).wait()
        win_bf[0] = stage_win[...].astype(_BF)
        win_load(1).start()
        wout_load(0).wait()
        wout_bf[0] = stage_wout[...].astype(_BF)
        wout_load(1).start()

        xall = layer(0, xall, last=False)
        xall = layer(1, xall, last=False)
        layer(2, xall, last=True)

    out_shape = jax.ShapeDtypeStruct((B, D), jnp.float32)
    return pl.pallas_call(
        body,
        out_shape=out_shape,
        in_specs=[pl.BlockSpec(memory_space=pltpu.VMEM)]
        + [pl.BlockSpec(memory_space=pltpu.MemorySpace.HBM)] * 6,
        out_specs=pl.BlockSpec(memory_space=pltpu.VMEM),
        scratch_shapes=[
            pltpu.VMEM((N_DEV, ROWS, D), _BF),
            pltpu.VMEM((N_DEV, ROWS, D), _BF),
            pltpu.VMEM((ROWS, D), _BF),
            pltpu.VMEM((N_DEV, ROWS, D), _BF),
            pltpu.VMEM((D, H), jnp.float32),
            pltpu.VMEM((H, D), jnp.float32),
            pltpu.VMEM((2, D, H), _BF),
            pltpu.VMEM((2, H, D), _BF),
            pltpu.SemaphoreType.DMA((2,)),
            pltpu.SemaphoreType.DMA((N_DEV,)),
            pltpu.SemaphoreType.DMA((N_DEV,)),
            pltpu.SemaphoreType.DMA((N_DEV,)),
            pltpu.SemaphoreType.DMA((N_DEV,)),
        ],
        compiler_params=pltpu.CompilerParams(
            has_side_effects=True,
            collective_id=0,
            vmem_limit_bytes=100 * 1024 * 1024,
        ),
    )(x, Win0, Wout0, Win1, Wout1, Win2, Wout2)
